# baseline (speedup 1.0000x reference)
import jax
import jax.numpy as jnp
from jax.experimental import pallas as pl
from jax.experimental.pallas import tpu as pltpu

_NCHUNK = 8
_CH = 1024


def _top2_chunk(x, kn):
    xss = jnp.sum(x * x, axis=1, keepdims=True)
    xn = x / jnp.maximum(jnp.sqrt(xss), 1e-12)

    sim = jax.lax.dot_general(
        xn, kn,
        dimension_numbers=(((1,), (1,)), ((), ())),
        preferred_element_type=jnp.float32,
    )

    m1 = jnp.max(sim, axis=1, keepdims=True)
    i1 = jnp.argmax(sim, axis=1, keepdims=True).astype(jnp.int32)
    iota = jax.lax.broadcasted_iota(jnp.int32, sim.shape, 1)
    sim2 = jnp.where(iota == i1, -jnp.inf, sim)
    m2 = jnp.max(sim2, axis=1, keepdims=True)
    i2 = jnp.argmax(sim2, axis=1, keepdims=True).astype(jnp.int32)

    e = jnp.exp(m2 - m1)
    denom = 1.0 + e
    w1 = 1.0 / denom
    w2 = e / denom
    return jnp.concatenate([i1, i2], axis=1), jnp.concatenate([w1, w2], axis=1)


def _router_kernel(x_hbm, k_ref, idx_ref, w_ref, buf, sem):
    copies = []
    for i in range(_NCHUNK):
        cp = pltpu.make_async_copy(
            x_hbm.at[pl.ds(i * _CH, _CH), :], buf.at[i], sem.at[i]
        )
        cp.start()
        copies.append(cp)

    keys = k_ref[...]
    kss = jnp.sum(keys * keys, axis=1, keepdims=True)
    kn = keys / jnp.maximum(jnp.sqrt(kss), 1e-12)

    for i in range(_NCHUNK):
        copies[i].wait()
        idx, w = _top2_chunk(buf[i], kn)
        idx_ref[pl.ds(i * _CH, _CH), :] = idx
        w_ref[pl.ds(i * _CH, _CH), :] = w


@jax.jit
def kernel(task_embedding, prompt_key):
    M, D = task_embedding.shape
    E = prompt_key.shape[0]
    idx, w = pl.pallas_call(
        _router_kernel,
        in_specs=[
            pl.BlockSpec(memory_space=pltpu.MemorySpace.HBM),
            pl.BlockSpec(memory_space=pltpu.MemorySpace.VMEM),
        ],
        out_specs=[
            pl.BlockSpec(memory_space=pltpu.MemorySpace.VMEM),
            pl.BlockSpec(memory_space=pltpu.MemorySpace.VMEM),
        ],
        out_shape=[
            jax.ShapeDtypeStruct((M, 2), jnp.int32),
            jax.ShapeDtypeStruct((M, 2), jnp.float32),
        ],
        scratch_shapes=[
            pltpu.VMEM((_NCHUNK, _CH, D), jnp.float32),
            pltpu.SemaphoreType.DMA((_NCHUNK,)),
        ],
    )(task_embedding, prompt_key)
    return idx, w


# speedup vs baseline: 1.1791x; 1.1791x over previous
import jax
import jax.numpy as jnp
from jax.experimental import pallas as pl

_BM = 4096


def _router_block(x_ref, k_ref, idx_ref, w_ref):
    keys = k_ref[...]
    kss = jnp.sum(keys * keys, axis=1, keepdims=True)
    kn = keys / jnp.maximum(jnp.sqrt(kss), 1e-12)

    x = x_ref[...]
    xss = jnp.sum(x * x, axis=1, keepdims=True)
    xn = x / jnp.maximum(jnp.sqrt(xss), 1e-12)

    sim = jax.lax.dot_general(
        xn, kn,
        dimension_numbers=(((1,), (1,)), ((), ())),
        preferred_element_type=jnp.float32,
    )

    m1 = jnp.max(sim, axis=1, keepdims=True)
    i1 = jnp.argmax(sim, axis=1, keepdims=True).astype(jnp.int32)
    iota = jax.lax.broadcasted_iota(jnp.int32, sim.shape, 1)
    sim2 = jnp.where(iota == i1, -jnp.inf, sim)
    m2 = jnp.max(sim2, axis=1, keepdims=True)
    i2 = jnp.argmax(sim2, axis=1, keepdims=True).astype(jnp.int32)

    e = jnp.exp(m2 - m1)
    denom = 1.0 + e
    w1 = 1.0 / denom
    w2 = e / denom

    idx_ref[...] = jnp.concatenate([i1, i2], axis=1)
    w_ref[...] = jnp.concatenate([w1, w2], axis=1)


@jax.jit
def kernel(task_embedding, prompt_key):
    M, D = task_embedding.shape
    E = prompt_key.shape[0]
    idx, w = pl.pallas_call(
        _router_block,
        grid=(M // _BM,),
        in_specs=[
            pl.BlockSpec((_BM, D), lambda i: (i, 0)),
            pl.BlockSpec((E, D), lambda i: (0, 0)),
        ],
        out_specs=[
            pl.BlockSpec((_BM, 2), lambda i: (i, 0)),
            pl.BlockSpec((_BM, 2), lambda i: (i, 0)),
        ],
        out_shape=[
            jax.ShapeDtypeStruct((M, 2), jnp.int32),
            jax.ShapeDtypeStruct((M, 2), jnp.float32),
        ],
    )(task_embedding, prompt_key)
    return idx, w
